# baseline (speedup 1.0000x reference)
import jax
import jax.numpy as jnp
from jax.experimental import pallas as pl
from jax.experimental.pallas import tpu as pltpu
from jax.experimental.pallas import tpu_sc as plsc

_E = 8
_D = 768
_H = 4 * _D
_T = 2048
_P = 2 * _T
_TM = 256
_NT = 24
_NTP = 32
_ROWS = _NT * _TM
_CS = 512

_DH = _D // 2
_DW = _D // 2
_GW = 128


def _router_body(x_ref, wg_ref, bg_ref,
                 pos0_ref, pos1_ref, g0_ref, g1_ref, te_ref,
                 rmap_ref, nu_ref):
    x = x_ref[...]
    logits = jnp.dot(x, wg_ref[...], preferred_element_type=jnp.float32)
    logits = logits + bg_ref[...]
    col = jax.lax.broadcasted_iota(jnp.int32, (_T, _E), 1)

    m1 = jnp.max(logits, axis=1, keepdims=True)
    idx1 = jnp.min(jnp.where(logits == m1, col, _E), axis=1, keepdims=True)
    oh1 = col == idx1
    masked = jnp.where(oh1, -jnp.inf, logits)
    m2 = jnp.max(masked, axis=1, keepdims=True)
    idx2 = jnp.min(jnp.where(masked == m2, col, _E), axis=1, keepdims=True)
    oh2 = col == idx2

    e21 = jnp.exp(m2 - m1)
    g0_ref[...] = 1.0 / (1.0 + e21)
    g1_ref[...] = e21 / (1.0 + e21)

    o1 = oh1.astype(jnp.float32)
    o2 = oh2.astype(jnp.float32)

    row = jax.lax.broadcasted_iota(jnp.int32, (_CS, _CS), 0)
    colr = jax.lax.broadcasted_iota(jnp.int32, (_CS, _CS), 1)
    stl = (colr < row).astype(jnp.float32)
    run = jnp.zeros((1, _E), jnp.float32)
    ranks = []
    for onehot in (o1, o2):
        rblocks = []
        for b in range(_T // _CS):
            ob = jax.lax.slice(onehot, (b * _CS, 0), ((b + 1) * _CS, _E))
            rblocks.append(
                jnp.dot(stl, ob, preferred_element_type=jnp.float32) + run)
            run = run + jnp.sum(ob, axis=0, keepdims=True)
        ranks.append(jnp.concatenate(rblocks, axis=0))
    rank1, rank2 = ranks
    counts = run

    pc = jnp.ceil(counts / _TM) * _TM
    er = jax.lax.broadcasted_iota(jnp.int32, (_E, _E), 0)
    ec = jax.lax.broadcasted_iota(jnp.int32, (_E, _E), 1)
    excl = (er < ec).astype(jnp.float32)
    poff = jnp.dot(pc, excl, preferred_element_type=jnp.float32)

    pos0 = jnp.sum((rank1 + poff) * o1, axis=1, keepdims=True)
    pos1 = jnp.sum((rank2 + poff) * o2, axis=1, keepdims=True)
    pos0_ref[...] = pos0.astype(jnp.int32)
    pos1_ref[...] = pos1.astype(jnp.int32)

    tend = (poff + pc) / _TM
    eye = (er == ec).astype(jnp.float32)
    tend_col = jnp.sum(jnp.broadcast_to(tend, (_E, _E)) * eye,
                       axis=1, keepdims=True)
    tid = jax.lax.broadcasted_iota(jnp.int32, (_E, _NTP), 1).astype(jnp.float32)
    ind = (tend_col <= tid).astype(jnp.int32)
    te = jnp.sum(ind, axis=0, keepdims=True)
    erow = jax.lax.broadcasted_iota(jnp.int32, (1, _E), 1)
    last_e = jnp.max(jnp.where(counts > 0, erow, 0), axis=1, keepdims=True)
    te_ref[...] = jnp.minimum(te, last_e)

    nu = (jnp.sum(pc, axis=1, keepdims=True) / _TM).astype(jnp.int32)
    nu_ref[...] = nu
    tid_i = jax.lax.broadcasted_iota(jnp.int32, (1, _NTP), 1)
    rmap_ref[...] = jnp.minimum(tid_i, nu - 1)


def _run_router(x2d, wg, bg2d):
    out_shapes = (
        jax.ShapeDtypeStruct((_T, 1), jnp.int32),
        jax.ShapeDtypeStruct((_T, 1), jnp.int32),
        jax.ShapeDtypeStruct((_T, 1), jnp.float32),
        jax.ShapeDtypeStruct((_T, 1), jnp.float32),
        jax.ShapeDtypeStruct((1, _NTP), jnp.int32),
        jax.ShapeDtypeStruct((1, _NTP), jnp.int32),
        jax.ShapeDtypeStruct((1, 1), jnp.int32),
    )
    return pl.pallas_call(
        _router_body,
        out_shape=out_shapes,
    )(x2d, wg, bg2d)


def _ffn_body(te_ref, rmap_ref, nu_ref,
              xs_ref, w1_ref, b1_ref, w2_ref, b2_ref, o_ref):
    del te_ref, rmap_ref

    @pl.when(pl.program_id(0) < nu_ref[0])
    def _():
        h = jnp.dot(xs_ref[...], w1_ref[0],
                    preferred_element_type=jnp.float32) + b1_ref[0]
        h = jnp.maximum(h, 0.0)
        o_ref[...] = jnp.dot(h.astype(jnp.bfloat16), w2_ref[0],
                             preferred_element_type=jnp.float32) + b2_ref[0]


def _run_ffn(te, rmap, nu, xs, w1, b13, w2, b23):
    grid_spec = pltpu.PrefetchScalarGridSpec(
        num_scalar_prefetch=3,
        grid=(_NT,),
        in_specs=[
            pl.BlockSpec((_TM, _D), lambda i, te, rm, nu: (rm[i], 0)),
            pl.BlockSpec((1, _D, _H), lambda i, te, rm, nu: (te[i], 0, 0)),
            pl.BlockSpec((1, 1, _H), lambda i, te, rm, nu: (te[i], 0, 0)),
            pl.BlockSpec((1, _H, _D), lambda i, te, rm, nu: (te[i], 0, 0)),
            pl.BlockSpec((1, 1, _D), lambda i, te, rm, nu: (te[i], 0, 0)),
        ],
        out_specs=pl.BlockSpec((_TM, _D), lambda i, te, rm, nu: (rm[i], 0)),
    )
    return pl.pallas_call(
        _ffn_body,
        grid_spec=grid_spec,
        out_shape=jax.ShapeDtypeStruct((_ROWS, _D), jnp.float32),
        compiler_params=pltpu.CompilerParams(vmem_limit_bytes=60 * 2**20),
    )(te, rmap, nu, xs, w1, b13, w2, b23)


def _run_dispatch(xx, ii):
    mesh = plsc.VectorSubcoreMesh(core_axis_name="core",
                                  subcore_axis_name="subcore")

    @pl.kernel(out_type=jax.ShapeDtypeStruct((_ROWS, _DW), jnp.int32),
               mesh=mesh)
    def dispatch(x_hbm, i_hbm, o_hbm):
        def body(x_vmem, i_vmem):
            pltpu.sync_copy(x_vmem, o_hbm.at[i_vmem.at[0]])

        pltpu.emit_pipeline(
            body,
            grid=(2 * _T // _GW,),
            in_specs=[
                pl.BlockSpec((_GW, _DW), lambda i: (i, 0)),
                pl.BlockSpec((1, _GW), lambda i: (0, i)),
            ],
            out_specs=[],
            core_axis_name=("core", "subcore"),
            dimension_semantics=(pltpu.PARALLEL,),
        )(x_hbm, i_hbm)

    return dispatch(xx, ii)


def _run_combine_gather(ysh, ii):
    mesh = plsc.VectorSubcoreMesh(core_axis_name="core",
                                  subcore_axis_name="subcore")

    @pl.kernel(out_type=jax.ShapeDtypeStruct((4 * _T, _DH), jnp.float32),
               mesh=mesh)
    def combine(ys_hbm, i_hbm, o_hbm):
        def body(i_vmem, o_vmem):
            pltpu.sync_copy(ys_hbm.at[i_vmem.at[0]], o_vmem)

        pltpu.emit_pipeline(
            body,
            grid=(4 * _T // _GW,),
            in_specs=[pl.BlockSpec((1, _GW), lambda i: (0, i))],
            out_specs=[pl.BlockSpec((_GW, _DH), lambda i: (i, 0))],
            core_axis_name=("core", "subcore"),
            dimension_semantics=(pltpu.PARALLEL,),
        )(i_hbm, o_hbm)

    return combine(ysh, ii)


def _wadd_body(y0_ref, y1_ref, g0_ref, g1_ref, o_ref):
    o_ref[...] = g0_ref[...] * y0_ref[...] + g1_ref[...] * y1_ref[...]


def _run_wadd(y0, y1, g0, g1):
    grid = (_T // _TM,)
    return pl.pallas_call(
        _wadd_body,
        grid=grid,
        in_specs=[
            pl.BlockSpec((_TM, _D), lambda i: (i, 0)),
            pl.BlockSpec((_TM, _D), lambda i: (i, 0)),
            pl.BlockSpec((_TM, 1), lambda i: (i, 0)),
            pl.BlockSpec((_TM, 1), lambda i: (i, 0)),
        ],
        out_specs=pl.BlockSpec((_TM, _D), lambda i: (i, 0)),
        out_shape=jax.ShapeDtypeStruct((_T, _D), jnp.float32),
    )(y0, y1, g0, g1)


def kernel(x, Wg, bg, W1, b1, W2, b2):
    b, s, d = x.shape
    x2d = x.reshape(_T, _D)
    bg2d = bg.reshape(1, _E)
    b13 = b1.reshape(_E, 1, _H)
    b23 = b2.reshape(_E, 1, _D)

    pos0, pos1, g0, g1, te, rmap, nu = _run_router(x2d, Wg, bg2d)

    xw = jax.lax.bitcast_convert_type(
        x2d.astype(jnp.bfloat16).reshape(_T, _DW, 2), jnp.int32)
    xx = jnp.concatenate([xw, xw], axis=0)
    ii = jnp.concatenate([pos0, pos1], axis=0).reshape(1, 2 * _T)
    xsw = _run_dispatch(xx, ii)
    xs = jax.lax.bitcast_convert_type(xsw, jnp.bfloat16).reshape(_ROWS, _D)

    ys = _run_ffn(te.reshape(_NTP), rmap.reshape(_NTP), nu.reshape(1),
                  xs, W1.astype(jnp.bfloat16), b13, W2.astype(jnp.bfloat16),
                  b23)

    i0 = jnp.concatenate([pos0 * 2, pos0 * 2 + 1], axis=1).reshape(1, 2 * _T)
    i1 = jnp.concatenate([pos1 * 2, pos1 * 2 + 1], axis=1).reshape(1, 2 * _T)
    iic = jnp.concatenate([i0, i1], axis=1)
    yy = _run_combine_gather(ys.reshape(2 * _ROWS, _DH), iic)
    y0 = yy[:2 * _T].reshape(_T, _D)
    y1 = yy[2 * _T:].reshape(_T, _D)
    out = _run_wadd(y0, y1, g0, g1)
    return out.reshape(b, s, d)


# speedup vs baseline: 1.6530x; 1.6530x over previous
import jax
import jax.numpy as jnp
from jax.experimental import pallas as pl
from jax.experimental.pallas import tpu as pltpu
from jax.experimental.pallas import tpu_sc as plsc

_E = 8
_D = 768
_H = 4 * _D
_T = 2048
_P = 2 * _T
_TM = 256
_NT = 24
_NTP = 32
_ROWS = _NT * _TM
_CS = 512

_DH = _D // 2
_GW = 128


def _router_body(x_ref, wg_ref, bg_ref,
                 pos0_ref, pos1_ref, g0_ref, g1_ref, te_ref,
                 rmap_ref, nu_ref):
    x = x_ref[...]
    logits = jnp.dot(x, wg_ref[...], preferred_element_type=jnp.float32)
    logits = logits + bg_ref[...]
    col = jax.lax.broadcasted_iota(jnp.int32, (_T, _E), 1)

    m1 = jnp.max(logits, axis=1, keepdims=True)
    idx1 = jnp.min(jnp.where(logits == m1, col, _E), axis=1, keepdims=True)
    oh1 = col == idx1
    masked = jnp.where(oh1, -jnp.inf, logits)
    m2 = jnp.max(masked, axis=1, keepdims=True)
    idx2 = jnp.min(jnp.where(masked == m2, col, _E), axis=1, keepdims=True)
    oh2 = col == idx2

    e21 = jnp.exp(m2 - m1)
    g0_ref[...] = 1.0 / (1.0 + e21)
    g1_ref[...] = e21 / (1.0 + e21)

    o1 = oh1.astype(jnp.float32)
    o2 = oh2.astype(jnp.float32)

    row = jax.lax.broadcasted_iota(jnp.int32, (_CS, _CS), 0)
    colr = jax.lax.broadcasted_iota(jnp.int32, (_CS, _CS), 1)
    stl = (colr < row).astype(jnp.float32)
    run = jnp.zeros((1, _E), jnp.float32)
    ranks = []
    for onehot in (o1, o2):
        rblocks = []
        for b in range(_T // _CS):
            ob = jax.lax.slice(onehot, (b * _CS, 0), ((b + 1) * _CS, _E))
            rblocks.append(
                jnp.dot(stl, ob, preferred_element_type=jnp.float32) + run)
            run = run + jnp.sum(ob, axis=0, keepdims=True)
        ranks.append(jnp.concatenate(rblocks, axis=0))
    rank1, rank2 = ranks
    counts = run

    pc = jnp.ceil(counts / _TM) * _TM
    er = jax.lax.broadcasted_iota(jnp.int32, (_E, _E), 0)
    ec = jax.lax.broadcasted_iota(jnp.int32, (_E, _E), 1)
    excl = (er < ec).astype(jnp.float32)
    poff = jnp.dot(pc, excl, preferred_element_type=jnp.float32)

    pos0 = jnp.sum((rank1 + poff) * o1, axis=1, keepdims=True)
    pos1 = jnp.sum((rank2 + poff) * o2, axis=1, keepdims=True)
    pos0_ref[...] = pos0.astype(jnp.int32)
    pos1_ref[...] = pos1.astype(jnp.int32)

    tend = (poff + pc) / _TM
    eye = (er == ec).astype(jnp.float32)
    tend_col = jnp.sum(jnp.broadcast_to(tend, (_E, _E)) * eye,
                       axis=1, keepdims=True)
    tid = jax.lax.broadcasted_iota(jnp.int32, (_E, _NTP), 1).astype(jnp.float32)
    ind = (tend_col <= tid).astype(jnp.int32)
    te = jnp.sum(ind, axis=0, keepdims=True)
    erow = jax.lax.broadcasted_iota(jnp.int32, (1, _E), 1)
    last_e = jnp.max(jnp.where(counts > 0, erow, 0), axis=1, keepdims=True)
    te_ref[...] = jnp.minimum(te, last_e)

    nu = (jnp.sum(pc, axis=1, keepdims=True) / _TM).astype(jnp.int32)
    nu_ref[...] = nu
    tid_i = jax.lax.broadcasted_iota(jnp.int32, (1, _NTP), 1)
    rmap_ref[...] = jnp.minimum(tid_i, nu - 1)


def _run_router(x2d, wg, bg2d):
    out_shapes = (
        jax.ShapeDtypeStruct((_T, 1), jnp.int32),
        jax.ShapeDtypeStruct((_T, 1), jnp.int32),
        jax.ShapeDtypeStruct((_T, 1), jnp.float32),
        jax.ShapeDtypeStruct((_T, 1), jnp.float32),
        jax.ShapeDtypeStruct((1, _NTP), jnp.int32),
        jax.ShapeDtypeStruct((1, _NTP), jnp.int32),
        jax.ShapeDtypeStruct((1, 1), jnp.int32),
    )
    return pl.pallas_call(
        _router_body,
        out_shape=out_shapes,
    )(x2d, wg, bg2d)


def _ffn_body(te_ref, rmap_ref, nu_ref,
              xs_ref, w1_ref, b1_ref, w2_ref, b2_ref, o_ref):
    del te_ref, rmap_ref

    @pl.when(pl.program_id(0) < nu_ref[0])
    def _():
        h = jnp.dot(xs_ref[...], w1_ref[0],
                    preferred_element_type=jnp.float32) + b1_ref[0]
        h = jnp.maximum(h, 0.0)
        o_ref[...] = jnp.dot(h, w2_ref[0],
                             preferred_element_type=jnp.float32) + b2_ref[0]


def _run_ffn(te, rmap, nu, xs, w1, b13, w2, b23):
    grid_spec = pltpu.PrefetchScalarGridSpec(
        num_scalar_prefetch=3,
        grid=(_NT,),
        in_specs=[
            pl.BlockSpec((_TM, _D), lambda i, te, rm, nu: (rm[i], 0)),
            pl.BlockSpec((1, _D, _H), lambda i, te, rm, nu: (te[i], 0, 0)),
            pl.BlockSpec((1, 1, _H), lambda i, te, rm, nu: (te[i], 0, 0)),
            pl.BlockSpec((1, _H, _D), lambda i, te, rm, nu: (te[i], 0, 0)),
            pl.BlockSpec((1, 1, _D), lambda i, te, rm, nu: (te[i], 0, 0)),
        ],
        out_specs=pl.BlockSpec((_TM, _D), lambda i, te, rm, nu: (rm[i], 0)),
    )
    return pl.pallas_call(
        _ffn_body,
        grid_spec=grid_spec,
        out_shape=jax.ShapeDtypeStruct((_ROWS, _D), jnp.float32),
        compiler_params=pltpu.CompilerParams(vmem_limit_bytes=60 * 2**20),
    )(te, rmap, nu, xs, w1, b13, w2, b23)


def _run_dispatch(xh, i0, i1):
    mesh = plsc.VectorSubcoreMesh(core_axis_name="core",
                                  subcore_axis_name="subcore")

    @pl.kernel(out_type=jax.ShapeDtypeStruct((2 * _ROWS, _DH), jnp.float32),
               mesh=mesh)
    def dispatch(x_hbm, i0_hbm, i1_hbm, o_hbm):
        def body(x_vmem, i_vmem):
            pltpu.sync_copy(x_vmem, o_hbm.at[i_vmem.at[0]])

        for ih in (i0_hbm, i1_hbm):
            pltpu.emit_pipeline(
                body,
                grid=(2 * _T // _GW,),
                in_specs=[
                    pl.BlockSpec((_GW, _DH), lambda i: (i, 0)),
                    pl.BlockSpec((1, _GW), lambda i: (0, i)),
                ],
                out_specs=[],
                core_axis_name=("core", "subcore"),
                dimension_semantics=(pltpu.PARALLEL,),
            )(x_hbm, ih)

    return dispatch(xh, i0, i1)


def _run_combine_gather(ysh, ii):
    mesh = plsc.VectorSubcoreMesh(core_axis_name="core",
                                  subcore_axis_name="subcore")

    @pl.kernel(out_type=jax.ShapeDtypeStruct((4 * _T, _DH), jnp.float32),
               mesh=mesh)
    def combine(ys_hbm, i_hbm, o_hbm):
        def body(i_vmem, o_vmem):
            pltpu.sync_copy(ys_hbm.at[i_vmem.at[0]], o_vmem)

        pltpu.emit_pipeline(
            body,
            grid=(4 * _T // _GW,),
            in_specs=[pl.BlockSpec((1, _GW), lambda i: (0, i))],
            out_specs=[pl.BlockSpec((_GW, _DH), lambda i: (i, 0))],
            core_axis_name=("core", "subcore"),
            dimension_semantics=(pltpu.PARALLEL,),
        )(i_hbm, o_hbm)

    return combine(ysh, ii)


def _wadd_body(y0_ref, y1_ref, g0_ref, g1_ref, o_ref):
    o_ref[...] = g0_ref[...] * y0_ref[...] + g1_ref[...] * y1_ref[...]


def _run_wadd(y0, y1, g0, g1):
    grid = (_T // _TM,)
    return pl.pallas_call(
        _wadd_body,
        grid=grid,
        in_specs=[
            pl.BlockSpec((_TM, _D), lambda i: (i, 0)),
            pl.BlockSpec((_TM, _D), lambda i: (i, 0)),
            pl.BlockSpec((_TM, 1), lambda i: (i, 0)),
            pl.BlockSpec((_TM, 1), lambda i: (i, 0)),
        ],
        out_specs=pl.BlockSpec((_TM, _D), lambda i: (i, 0)),
        out_shape=jax.ShapeDtypeStruct((_T, _D), jnp.float32),
    )(y0, y1, g0, g1)


def kernel(x, Wg, bg, W1, b1, W2, b2):
    b, s, d = x.shape
    x2d = x.reshape(_T, _D)
    bg2d = bg.reshape(1, _E)
    b13 = b1.reshape(_E, 1, _H)
    b23 = b2.reshape(_E, 1, _D)

    pos0, pos1, g0, g1, te, rmap, nu = _run_router(x2d, Wg, bg2d)

    i0 = jnp.concatenate([pos0 * 2, pos0 * 2 + 1], axis=1).reshape(1, 2 * _T)
    i1 = jnp.concatenate([pos1 * 2, pos1 * 2 + 1], axis=1).reshape(1, 2 * _T)

    xs = _run_dispatch(x2d.reshape(2 * _T, _DH), i0, i1).reshape(_ROWS, _D)
    ys = _run_ffn(te.reshape(_NTP), rmap.reshape(_NTP), nu.reshape(1),
                  xs, W1, b13, W2, b23)

    iic = jnp.concatenate([i0, i1], axis=1)
    yy = _run_combine_gather(ys.reshape(2 * _ROWS, _DH), iic)
    y0 = yy[:2 * _T].reshape(_T, _D)
    y1 = yy[2 * _T:].reshape(_T, _D)
    out = _run_wadd(y0, y1, g0, g1)
    return out.reshape(b, s, d)


# speedup vs baseline: 1.7344x; 1.0492x over previous
import jax
import jax.numpy as jnp
from jax.experimental import pallas as pl
from jax.experimental.pallas import tpu as pltpu
from jax.experimental.pallas import tpu_sc as plsc

_E = 8
_D = 768
_H = 4 * _D
_T = 2048
_P = 2 * _T
_TM = 256
_NT = 24
_NTP = 32
_ROWS = _NT * _TM
_CS = 512

_DH = _D // 2
_GW = 128


def _router_body(x_ref, wg_ref, bg_ref,
                 pos0_ref, pos1_ref, g0_ref, g1_ref, te_ref,
                 rmap_ref, nu_ref):
    x = x_ref[...]
    logits = jnp.dot(x, wg_ref[...], preferred_element_type=jnp.float32)
    logits = logits + bg_ref[...]
    col = jax.lax.broadcasted_iota(jnp.int32, (_T, _E), 1)

    m1 = jnp.max(logits, axis=1, keepdims=True)
    idx1 = jnp.min(jnp.where(logits == m1, col, _E), axis=1, keepdims=True)
    oh1 = col == idx1
    masked = jnp.where(oh1, -jnp.inf, logits)
    m2 = jnp.max(masked, axis=1, keepdims=True)
    idx2 = jnp.min(jnp.where(masked == m2, col, _E), axis=1, keepdims=True)
    oh2 = col == idx2

    e21 = jnp.exp(m2 - m1)
    g0_ref[...] = 1.0 / (1.0 + e21)
    g1_ref[...] = e21 / (1.0 + e21)

    o1 = oh1.astype(jnp.float32)
    o2 = oh2.astype(jnp.float32)

    row = jax.lax.broadcasted_iota(jnp.int32, (_CS, _CS), 0)
    colr = jax.lax.broadcasted_iota(jnp.int32, (_CS, _CS), 1)
    stl = (colr < row).astype(jnp.float32)
    run = jnp.zeros((1, _E), jnp.float32)
    ranks = []
    for onehot in (o1, o2):
        rblocks = []
        for b in range(_T // _CS):
            ob = jax.lax.slice(onehot, (b * _CS, 0), ((b + 1) * _CS, _E))
            rblocks.append(
                jnp.dot(stl, ob, preferred_element_type=jnp.float32) + run)
            run = run + jnp.sum(ob, axis=0, keepdims=True)
        ranks.append(jnp.concatenate(rblocks, axis=0))
    rank1, rank2 = ranks
    counts = run

    pc = jnp.ceil(counts / _TM) * _TM
    er = jax.lax.broadcasted_iota(jnp.int32, (_E, _E), 0)
    ec = jax.lax.broadcasted_iota(jnp.int32, (_E, _E), 1)
    excl = (er < ec).astype(jnp.float32)
    poff = jnp.dot(pc, excl, preferred_element_type=jnp.float32)

    pos0 = jnp.sum((rank1 + poff) * o1, axis=1, keepdims=True)
    pos1 = jnp.sum((rank2 + poff) * o2, axis=1, keepdims=True)
    pos0_ref[...] = pos0.astype(jnp.int32)
    pos1_ref[...] = pos1.astype(jnp.int32)

    tend = (poff + pc) / _TM
    eye = (er == ec).astype(jnp.float32)
    tend_col = jnp.sum(jnp.broadcast_to(tend, (_E, _E)) * eye,
                       axis=1, keepdims=True)
    tid = jax.lax.broadcasted_iota(jnp.int32, (_E, _NTP), 1).astype(jnp.float32)
    ind = (tend_col <= tid).astype(jnp.int32)
    te = jnp.sum(ind, axis=0, keepdims=True)
    erow = jax.lax.broadcasted_iota(jnp.int32, (1, _E), 1)
    last_e = jnp.max(jnp.where(counts > 0, erow, 0), axis=1, keepdims=True)
    te_ref[...] = jnp.minimum(te, last_e)

    nu = (jnp.sum(pc, axis=1, keepdims=True) / _TM).astype(jnp.int32)
    nu_ref[...] = nu
    tid_i = jax.lax.broadcasted_iota(jnp.int32, (1, _NTP), 1)
    rmap_ref[...] = jnp.minimum(tid_i, nu - 1)


def _run_router(x2d, wg, bg2d):
    out_shapes = (
        jax.ShapeDtypeStruct((_T, 1), jnp.int32),
        jax.ShapeDtypeStruct((_T, 1), jnp.int32),
        jax.ShapeDtypeStruct((_T, 1), jnp.float32),
        jax.ShapeDtypeStruct((_T, 1), jnp.float32),
        jax.ShapeDtypeStruct((1, _NTP), jnp.int32),
        jax.ShapeDtypeStruct((1, _NTP), jnp.int32),
        jax.ShapeDtypeStruct((1, 1), jnp.int32),
    )
    return pl.pallas_call(
        _router_body,
        out_shape=out_shapes,
    )(x2d, wg, bg2d)


def _ffn_body(te_ref, rmap_ref, nu_ref,
              x_ref, p0_ref, p1_ref, w1_ref, b1_ref, w2_ref, b2_ref, o_ref):
    del te_ref

    @pl.when(pl.program_id(0) < nu_ref[0])
    def _():
        r0 = rmap_ref[pl.program_id(0)] * _TM
        rid = jax.lax.broadcasted_iota(jnp.int32, (_TM, _T), 0) + r0
        sel = jnp.logical_or(p0_ref[...] == rid, p1_ref[...] == rid)
        xs = jnp.dot(sel.astype(jnp.float32), x_ref[...],
                     preferred_element_type=jnp.float32)
        h = jnp.dot(xs, w1_ref[0],
                    preferred_element_type=jnp.float32) + b1_ref[0]
        h = jnp.maximum(h, 0.0)
        o_ref[...] = jnp.dot(h, w2_ref[0],
                             preferred_element_type=jnp.float32) + b2_ref[0]


def _run_ffn(te, rmap, nu, x2d, p0, p1, w1, b13, w2, b23):
    grid_spec = pltpu.PrefetchScalarGridSpec(
        num_scalar_prefetch=3,
        grid=(_NT,),
        in_specs=[
            pl.BlockSpec((_T, _D), lambda i, te, rm, nu: (0, 0)),
            pl.BlockSpec((1, _T), lambda i, te, rm, nu: (0, 0)),
            pl.BlockSpec((1, _T), lambda i, te, rm, nu: (0, 0)),
            pl.BlockSpec((1, _D, _H), lambda i, te, rm, nu: (te[i], 0, 0)),
            pl.BlockSpec((1, 1, _H), lambda i, te, rm, nu: (te[i], 0, 0)),
            pl.BlockSpec((1, _H, _D), lambda i, te, rm, nu: (te[i], 0, 0)),
            pl.BlockSpec((1, 1, _D), lambda i, te, rm, nu: (te[i], 0, 0)),
        ],
        out_specs=pl.BlockSpec((_TM, _D), lambda i, te, rm, nu: (rm[i], 0)),
    )
    return pl.pallas_call(
        _ffn_body,
        grid_spec=grid_spec,
        out_shape=jax.ShapeDtypeStruct((_ROWS, _D), jnp.float32),
        compiler_params=pltpu.CompilerParams(vmem_limit_bytes=64 * 2**20),
    )(te, rmap, nu, x2d, p0, p1, w1, b13, w2, b23)


def _run_combine_gather(ysh, ii):
    mesh = plsc.VectorSubcoreMesh(core_axis_name="core",
                                  subcore_axis_name="subcore")

    @pl.kernel(out_type=jax.ShapeDtypeStruct((4 * _T, _DH), jnp.float32),
               mesh=mesh)
    def combine(ys_hbm, i_hbm, o_hbm):
        def body(i_vmem, o_vmem):
            pltpu.sync_copy(ys_hbm.at[i_vmem.at[0]], o_vmem)

        pltpu.emit_pipeline(
            body,
            grid=(4 * _T // _GW,),
            in_specs=[pl.BlockSpec((1, _GW), lambda i: (0, i))],
            out_specs=[pl.BlockSpec((_GW, _DH), lambda i: (i, 0))],
            core_axis_name=("core", "subcore"),
            dimension_semantics=(pltpu.PARALLEL,),
        )(i_hbm, o_hbm)

    return combine(ysh, ii)


def _wadd_body(y0_ref, y1_ref, g0_ref, g1_ref, o_ref):
    o_ref[...] = g0_ref[...] * y0_ref[...] + g1_ref[...] * y1_ref[...]


def _run_wadd(y0, y1, g0, g1):
    grid = (_T // _TM,)
    return pl.pallas_call(
        _wadd_body,
        grid=grid,
        in_specs=[
            pl.BlockSpec((_TM, _D), lambda i: (i, 0)),
            pl.BlockSpec((_TM, _D), lambda i: (i, 0)),
            pl.BlockSpec((_TM, 1), lambda i: (i, 0)),
            pl.BlockSpec((_TM, 1), lambda i: (i, 0)),
        ],
        out_specs=pl.BlockSpec((_TM, _D), lambda i: (i, 0)),
        out_shape=jax.ShapeDtypeStruct((_T, _D), jnp.float32),
    )(y0, y1, g0, g1)


def kernel(x, Wg, bg, W1, b1, W2, b2):
    b, s, d = x.shape
    x2d = x.reshape(_T, _D)
    bg2d = bg.reshape(1, _E)
    b13 = b1.reshape(_E, 1, _H)
    b23 = b2.reshape(_E, 1, _D)

    pos0, pos1, g0, g1, te, rmap, nu = _run_router(x2d, Wg, bg2d)

    ys = _run_ffn(te.reshape(_NTP), rmap.reshape(_NTP), nu.reshape(1),
                  x2d, pos0.reshape(1, _T), pos1.reshape(1, _T),
                  W1, b13, W2, b23)

    i0 = jnp.concatenate([pos0 * 2, pos0 * 2 + 1], axis=1).reshape(1, 2 * _T)
    i1 = jnp.concatenate([pos1 * 2, pos1 * 2 + 1], axis=1).reshape(1, 2 * _T)
    iic = jnp.concatenate([i0, i1], axis=1)
    yy = _run_combine_gather(ys.reshape(2 * _ROWS, _DH), iic)
    y0 = yy[:2 * _T].reshape(_T, _D)
    y1 = yy[2 * _T:].reshape(_T, _D)
    out = _run_wadd(y0, y1, g0, g1)
    return out.reshape(b, s, d)


# speedup vs baseline: 2.5773x; 1.4860x over previous
import jax
import jax.numpy as jnp
from jax.experimental import pallas as pl
from jax.experimental.pallas import tpu as pltpu
from jax.experimental.pallas import tpu_sc as plsc

_E = 8
_D = 768
_H = 4 * _D
_T = 2048
_P = 2 * _T
_TM = 256
_NT = 24
_NTP = 32
_ROWS = _NT * _TM
_CS = 512

_DH = _D // 2
_GW = 128


def _router_body(x_ref, wg_ref, bg_ref,
                 pos0_ref, pos1_ref, g0_ref, g1_ref, te_ref,
                 rmap_ref, nu_ref):
    x = x_ref[...]
    logits = jnp.dot(x, wg_ref[...], preferred_element_type=jnp.float32)
    logits = logits + bg_ref[...]
    col = jax.lax.broadcasted_iota(jnp.int32, (_T, _E), 1)

    m1 = jnp.max(logits, axis=1, keepdims=True)
    idx1 = jnp.min(jnp.where(logits == m1, col, _E), axis=1, keepdims=True)
    oh1 = col == idx1
    masked = jnp.where(oh1, -jnp.inf, logits)
    m2 = jnp.max(masked, axis=1, keepdims=True)
    idx2 = jnp.min(jnp.where(masked == m2, col, _E), axis=1, keepdims=True)
    oh2 = col == idx2

    e21 = jnp.exp(m2 - m1)
    g0_ref[...] = 1.0 / (1.0 + e21)
    g1_ref[...] = e21 / (1.0 + e21)

    o1 = oh1.astype(jnp.float32)
    o2 = oh2.astype(jnp.float32)

    row = jax.lax.broadcasted_iota(jnp.int32, (_CS, _CS), 0)
    colr = jax.lax.broadcasted_iota(jnp.int32, (_CS, _CS), 1)
    stl = (colr < row).astype(jnp.float32)
    run = jnp.zeros((1, _E), jnp.float32)
    ranks = []
    for onehot in (o1, o2):
        rblocks = []
        for b in range(_T // _CS):
            ob = jax.lax.slice(onehot, (b * _CS, 0), ((b + 1) * _CS, _E))
            rblocks.append(
                jnp.dot(stl, ob, preferred_element_type=jnp.float32) + run)
            run = run + jnp.sum(ob, axis=0, keepdims=True)
        ranks.append(jnp.concatenate(rblocks, axis=0))
    rank1, rank2 = ranks
    counts = run

    pc = jnp.ceil(counts / _TM) * _TM
    er = jax.lax.broadcasted_iota(jnp.int32, (_E, _E), 0)
    ec = jax.lax.broadcasted_iota(jnp.int32, (_E, _E), 1)
    excl = (er < ec).astype(jnp.float32)
    poff = jnp.dot(pc, excl, preferred_element_type=jnp.float32)

    pos0 = jnp.sum((rank1 + poff) * o1, axis=1, keepdims=True)
    pos1 = jnp.sum((rank2 + poff) * o2, axis=1, keepdims=True)
    pos0_ref[...] = pos0.astype(jnp.int32)
    pos1_ref[...] = pos1.astype(jnp.int32)

    tend = (poff + pc) / _TM
    eye = (er == ec).astype(jnp.float32)
    tend_col = jnp.sum(jnp.broadcast_to(tend, (_E, _E)) * eye,
                       axis=1, keepdims=True)
    tid = jax.lax.broadcasted_iota(jnp.int32, (_E, _NTP), 1).astype(jnp.float32)
    ind = (tend_col <= tid).astype(jnp.int32)
    te = jnp.sum(ind, axis=0, keepdims=True)
    erow = jax.lax.broadcasted_iota(jnp.int32, (1, _E), 1)
    last_e = jnp.max(jnp.where(counts > 0, erow, 0), axis=1, keepdims=True)
    te_ref[...] = jnp.minimum(te, last_e)

    nu = (jnp.sum(pc, axis=1, keepdims=True) / _TM).astype(jnp.int32)
    nu_ref[...] = nu
    tid_i = jax.lax.broadcasted_iota(jnp.int32, (1, _NTP), 1)
    rmap_ref[...] = jnp.minimum(tid_i, nu - 1)


def _run_router(x2d, wg, bg2d):
    out_shapes = (
        jax.ShapeDtypeStruct((_T, 1), jnp.int32),
        jax.ShapeDtypeStruct((_T, 1), jnp.int32),
        jax.ShapeDtypeStruct((_T, 1), jnp.float32),
        jax.ShapeDtypeStruct((_T, 1), jnp.float32),
        jax.ShapeDtypeStruct((1, _NTP), jnp.int32),
        jax.ShapeDtypeStruct((1, _NTP), jnp.int32),
        jax.ShapeDtypeStruct((1, 1), jnp.int32),
    )
    return pl.pallas_call(
        _router_body,
        out_shape=out_shapes,
    )(x2d, wg, bg2d)


def _ffn_body(te_ref, rmap_ref, nu_ref,
              x_ref, p0r_ref, p1r_ref, p0c_ref, p1c_ref, g0_ref, g1_ref,
              w1_ref, b1_ref, w2_ref, b2_ref, o_ref):
    del te_ref
    i = pl.program_id(0)

    @pl.when(i < nu_ref[0])
    def _():
        r0 = rmap_ref[i] * _TM
        rid = jax.lax.broadcasted_iota(jnp.int32, (_TM, _T), 0) + r0
        sel = jnp.logical_or(p0r_ref[...] == rid, p1r_ref[...] == rid)
        xs = jnp.dot(sel.astype(jnp.float32), x_ref[...],
                     preferred_element_type=jnp.float32)
        h = jnp.dot(xs, w1_ref[0],
                    preferred_element_type=jnp.float32) + b1_ref[0]
        h = jnp.maximum(h, 0.0)
        y = jnp.dot(h, w2_ref[0],
                    preferred_element_type=jnp.float32) + b2_ref[0]

        ridr = jax.lax.broadcasted_iota(jnp.int32, (_T, _TM), 1) + r0
        sgt = (jnp.where(p0c_ref[...] == ridr, g0_ref[...], 0.0)
               + jnp.where(p1c_ref[...] == ridr, g1_ref[...], 0.0))
        contrib = jnp.dot(sgt, y, preferred_element_type=jnp.float32)

        @pl.when(i == 0)
        def _():
            o_ref[...] = contrib

        @pl.when(i > 0)
        def _():
            o_ref[...] += contrib


def _run_ffn(te, rmap, nu, x2d, p0r, p1r, p0c, p1c, g0, g1,
             w1, b13, w2, b23):
    grid_spec = pltpu.PrefetchScalarGridSpec(
        num_scalar_prefetch=3,
        grid=(_NT,),
        in_specs=[
            pl.BlockSpec((_T, _D), lambda i, te, rm, nu: (0, 0)),
            pl.BlockSpec((1, _T), lambda i, te, rm, nu: (0, 0)),
            pl.BlockSpec((1, _T), lambda i, te, rm, nu: (0, 0)),
            pl.BlockSpec((_T, 1), lambda i, te, rm, nu: (0, 0)),
            pl.BlockSpec((_T, 1), lambda i, te, rm, nu: (0, 0)),
            pl.BlockSpec((_T, 1), lambda i, te, rm, nu: (0, 0)),
            pl.BlockSpec((_T, 1), lambda i, te, rm, nu: (0, 0)),
            pl.BlockSpec((1, _D, _H), lambda i, te, rm, nu: (te[i], 0, 0)),
            pl.BlockSpec((1, 1, _H), lambda i, te, rm, nu: (te[i], 0, 0)),
            pl.BlockSpec((1, _H, _D), lambda i, te, rm, nu: (te[i], 0, 0)),
            pl.BlockSpec((1, 1, _D), lambda i, te, rm, nu: (te[i], 0, 0)),
        ],
        out_specs=pl.BlockSpec((_T, _D), lambda i, te, rm, nu: (0, 0)),
    )
    return pl.pallas_call(
        _ffn_body,
        grid_spec=grid_spec,
        out_shape=jax.ShapeDtypeStruct((_T, _D), jnp.float32),
        compiler_params=pltpu.CompilerParams(vmem_limit_bytes=64 * 2**20),
    )(te, rmap, nu, x2d, p0r, p1r, p0c, p1c, g0, g1, w1, b13, w2, b23)


def _run_combine_gather(ysh, ii):
    mesh = plsc.VectorSubcoreMesh(core_axis_name="core",
                                  subcore_axis_name="subcore")

    @pl.kernel(out_type=jax.ShapeDtypeStruct((4 * _T, _DH), jnp.float32),
               mesh=mesh)
    def combine(ys_hbm, i_hbm, o_hbm):
        def body(i_vmem, o_vmem):
            pltpu.sync_copy(ys_hbm.at[i_vmem.at[0]], o_vmem)

        pltpu.emit_pipeline(
            body,
            grid=(4 * _T // _GW,),
            in_specs=[pl.BlockSpec((1, _GW), lambda i: (0, i))],
            out_specs=[pl.BlockSpec((_GW, _DH), lambda i: (i, 0))],
            core_axis_name=("core", "subcore"),
            dimension_semantics=(pltpu.PARALLEL,),
        )(i_hbm, o_hbm)

    return combine(ysh, ii)


def _wadd_body(y0_ref, y1_ref, g0_ref, g1_ref, o_ref):
    o_ref[...] = g0_ref[...] * y0_ref[...] + g1_ref[...] * y1_ref[...]


def _run_wadd(y0, y1, g0, g1):
    grid = (_T // _TM,)
    return pl.pallas_call(
        _wadd_body,
        grid=grid,
        in_specs=[
            pl.BlockSpec((_TM, _D), lambda i: (i, 0)),
            pl.BlockSpec((_TM, _D), lambda i: (i, 0)),
            pl.BlockSpec((_TM, 1), lambda i: (i, 0)),
            pl.BlockSpec((_TM, 1), lambda i: (i, 0)),
        ],
        out_specs=pl.BlockSpec((_TM, _D), lambda i: (i, 0)),
        out_shape=jax.ShapeDtypeStruct((_T, _D), jnp.float32),
    )(y0, y1, g0, g1)


def kernel(x, Wg, bg, W1, b1, W2, b2):
    b, s, d = x.shape
    x2d = x.reshape(_T, _D)
    bg2d = bg.reshape(1, _E)
    b13 = b1.reshape(_E, 1, _H)
    b23 = b2.reshape(_E, 1, _D)

    pos0, pos1, g0, g1, te, rmap, nu = _run_router(x2d, Wg, bg2d)

    out = _run_ffn(te.reshape(_NTP), rmap.reshape(_NTP), nu.reshape(1),
                   x2d, pos0.reshape(1, _T), pos1.reshape(1, _T),
                   pos0, pos1, g0, g1, W1, b13, W2, b23)
    return out.reshape(b, s, d)
